# Initial kernel scaffold; baseline (speedup 1.0000x reference)
import functools

import jax
import jax.numpy as jnp
from jax import lax
from jax.experimental import pallas as pl
from jax.experimental.pallas import tpu as pltpu
from jax.experimental.pallas import tpu_sc as plsc

VOCAB = 1000000
D = 32
B = 16384 * 50
NC, NS = 2, 16
NW = NC * NS
BPW = B // NW
C = 128
NCHUNK = BPW // C
NBUF = 4

_mesh = plsc.VectorSubcoreMesh(core_axis_name="c", subcore_axis_name="s")


@functools.partial(
    pl.kernel,
    out_type=jax.ShapeDtypeStruct((B, D), jnp.float32),
    mesh=_mesh,
    scratch_types=[
        pltpu.VMEM((BPW,), jnp.int32),
        pltpu.VMEM((NBUF, C, D), jnp.float32),
        pltpu.SemaphoreType.DMA((NBUF,)),
    ],
)
def _embed(x_hbm, table_hbm, out_hbm, idx_v, rows_v, gsem):
  wid = lax.axis_index("s") * NC + lax.axis_index("c")
  base = wid * BPW

  pltpu.sync_copy(x_hbm.at[pl.ds(base, BPW)], idx_v)

  def gather_start(b, j):
    pltpu.async_copy(
        table_hbm.at[idx_v.at[pl.ds(j * C, C)]], rows_v.at[b], gsem.at[b])

  def gather_wait(b, j):
    pltpu.make_async_copy(
        table_hbm.at[idx_v.at[pl.ds(j * C, C)]], rows_v.at[b],
        gsem.at[b]).wait()

  for b in range(NBUF):
    gather_start(b, b)

  @pl.loop(0, NCHUNK, step=NBUF)
  def _round(j0):
    for b in range(NBUF):
      j = j0 + b
      gather_wait(b, j)
      pltpu.sync_copy(rows_v.at[b], out_hbm.at[pl.ds(base + j * C, C)])
      nj = j + NBUF

      @pl.when(nj < NCHUNK)
      def _():
        gather_start(b, nj)


def kernel(x, table):
  flat = x.reshape(-1).astype(jnp.int32)
  out = _embed(flat, table)
  return out.reshape(x.shape[0], x.shape[1], D)


# speedup vs baseline: 1.1105x; 1.1105x over previous
import functools

import jax
import jax.numpy as jnp
from jax import lax
from jax.experimental import pallas as pl
from jax.experimental.pallas import tpu as pltpu
from jax.experimental.pallas import tpu_sc as plsc

VOCAB = 1000000
D = 32
B = 16384 * 50
NC, NS = 2, 16
NW = NC * NS
BPW = B // NW
C = 128
NCHUNK = BPW // C
NBUF = 4

_mesh = plsc.VectorSubcoreMesh(core_axis_name="c", subcore_axis_name="s")


@functools.partial(
    pl.kernel,
    out_type=jax.ShapeDtypeStruct((B, D), jnp.float32),
    mesh=_mesh,
    scratch_types=[
        pltpu.VMEM((BPW,), jnp.int32),
        pltpu.VMEM((NBUF, C, D), jnp.float32),
        pltpu.SemaphoreType.DMA((NBUF,)),
    ],
    compiler_params=pltpu.CompilerParams(use_tc_tiling_on_sc=False),
)
def _embed(x_hbm, table_hbm, out_hbm, idx_v, rows_v, gsem):
  wid = lax.axis_index("s") * NC + lax.axis_index("c")
  base = wid * BPW

  pltpu.sync_copy(x_hbm.at[pl.ds(base, BPW)], idx_v)

  def gather_start(b, j):
    pltpu.async_copy(
        table_hbm.at[idx_v.at[pl.ds(j * C, C)]], rows_v.at[b], gsem.at[b])

  def gather_wait(b, j):
    pltpu.make_async_copy(
        table_hbm.at[idx_v.at[pl.ds(j * C, C)]], rows_v.at[b],
        gsem.at[b]).wait()

  for b in range(NBUF):
    gather_start(b, b)

  @pl.loop(0, NCHUNK, step=NBUF)
  def _round(j0):
    for b in range(NBUF):
      j = j0 + b
      gather_wait(b, j)
      pltpu.sync_copy(rows_v.at[b], out_hbm.at[pl.ds(base + j * C, C)])
      nj = j + NBUF

      @pl.when(nj < NCHUNK)
      def _():
        gather_start(b, nj)


def kernel(x, table):
  flat = x.reshape(-1).astype(jnp.int32)
  out = _embed(flat, table)
  return out.reshape(x.shape[0], x.shape[1], D)


# speedup vs baseline: 1.1141x; 1.0032x over previous
import functools

import jax
import jax.numpy as jnp
from jax import lax
from jax.experimental import pallas as pl
from jax.experimental.pallas import tpu as pltpu
from jax.experimental.pallas import tpu_sc as plsc

VOCAB = 1000000
D = 32
B = 16384 * 50
NC, NS = 2, 16
NW = NC * NS
BPW = B // NW
C = 128
NCHUNK = BPW // C
NBUF = 8

_mesh = plsc.VectorSubcoreMesh(core_axis_name="c", subcore_axis_name="s")


@functools.partial(
    pl.kernel,
    out_type=jax.ShapeDtypeStruct((B, D), jnp.float32),
    mesh=_mesh,
    scratch_types=[
        pltpu.VMEM((BPW,), jnp.int32),
        pltpu.VMEM((NBUF, C, D), jnp.float32),
        pltpu.SemaphoreType.DMA((NBUF,)),
    ],
    compiler_params=pltpu.CompilerParams(use_tc_tiling_on_sc=False),
)
def _embed(x_hbm, table_hbm, out_hbm, idx_v, rows_v, gsem):
  wid = lax.axis_index("s") * NC + lax.axis_index("c")
  base = wid * BPW

  pltpu.sync_copy(x_hbm.at[pl.ds(base, BPW)], idx_v)

  def gather_start(b, j):
    pltpu.async_copy(
        table_hbm.at[idx_v.at[pl.ds(j * C, C)]], rows_v.at[b], gsem.at[b])

  def gather_wait(b, j):
    pltpu.make_async_copy(
        table_hbm.at[idx_v.at[pl.ds(j * C, C)]], rows_v.at[b],
        gsem.at[b]).wait()

  for b in range(NBUF):
    gather_start(b, b)

  @pl.loop(0, NCHUNK, step=NBUF)
  def _round(j0):
    for b in range(NBUF):
      j = j0 + b
      gather_wait(b, j)
      pltpu.sync_copy(rows_v.at[b], out_hbm.at[pl.ds(base + j * C, C)])
      nj = j + NBUF

      @pl.when(nj < NCHUNK)
      def _():
        gather_start(b, nj)


def kernel(x, table):
  flat = x.reshape(-1).astype(jnp.int32)
  out = _embed(flat, table)
  return out.reshape(x.shape[0], x.shape[1], D)


# speedup vs baseline: 1.7983x; 1.6141x over previous
import functools

import jax
import jax.numpy as jnp
from jax import lax
from jax.experimental import pallas as pl
from jax.experimental.pallas import tpu as pltpu
from jax.experimental.pallas import tpu_sc as plsc

VOCAB = 1000000
D = 32
BATCH = 16384
HIST = 50
NC, NS = 2, 16
NW = NC * NS
XPW = BATCH // NW
NBUF = 8

_mesh = plsc.VectorSubcoreMesh(core_axis_name="c", subcore_axis_name="s")


@functools.partial(
    pl.kernel,
    out_type=jax.ShapeDtypeStruct((BATCH, HIST, D), jnp.float32),
    mesh=_mesh,
    scratch_types=[
        pltpu.VMEM((XPW, HIST), jnp.int32),
        pltpu.VMEM((NBUF, HIST, D), jnp.float32),
        pltpu.SemaphoreType.DMA((NBUF,)),
    ],
    compiler_params=pltpu.CompilerParams(use_tc_tiling_on_sc=False),
)
def _embed(x_hbm, table_hbm, out3_hbm, idx_v, rows_v, gsem):
  wid = lax.axis_index("s") * NC + lax.axis_index("c")
  xbase = wid * XPW

  pltpu.sync_copy(x_hbm.at[pl.ds(xbase, XPW)], idx_v)

  def gather_start(b, j):
    pltpu.async_copy(
        table_hbm.at[idx_v.at[j]], rows_v.at[b], gsem.at[b])

  def gather_wait(b, j):
    pltpu.make_async_copy(
        table_hbm.at[idx_v.at[j]], rows_v.at[b], gsem.at[b]).wait()

  for b in range(NBUF):
    gather_start(b, b)

  @pl.loop(0, XPW, step=NBUF)
  def _round(j0):
    for b in range(NBUF):
      j = j0 + b
      gather_wait(b, j)
      pltpu.sync_copy(rows_v.at[b], out3_hbm.at[xbase + j])
      nj = j + NBUF

      @pl.when(nj < XPW)
      def _():
        gather_start(b, nj)


def kernel(x, table):
  return _embed(x.astype(jnp.int32), table)
